# baseline (speedup 1.0000x reference)
import functools

import jax
import jax.numpy as jnp
from jax import lax
from jax.experimental import pallas as pl
from jax.experimental.pallas import tpu as pltpu
from jax.experimental.pallas import tpu_sc as plsc

_L = 16


def _wsum(om, a, u, v):
    def term(s, w):
        if isinstance(w, float):
            if w == 0.0:
                return None
            if w == 1.0:
                return s
        return s * w
    t1, t2 = term(om, u), term(a, v)
    if t1 is None and t2 is None:
        return 0.0
    if t1 is None:
        return t2
    if t2 is None:
        return t1
    return t1 + t2


def _make_sc_eval(B, n_coef, C, n_knots):
    NC, NS = 2, 16
    NW = NC * NS
    assert B % NW == 0
    rows_per_w = B // NW
    chunk = 256
    assert rows_per_w % chunk == 0
    n_chunks = rows_per_w // chunk
    groups = chunk // _L
    n_dma = (chunk * 2) // 128
    n_blk = B * n_coef * C // 128

    mesh = plsc.VectorSubcoreMesh(core_axis_name="c", subcore_axis_name="s",
                                  num_cores=NC, num_subcores=NS)

    @functools.partial(
        pl.kernel,
        out_type=jax.ShapeDtypeStruct((B, C), jnp.float32),
        mesh=mesh,
        scratch_types=[
            pltpu.VMEM((chunk, n_knots), jnp.float32),
            pltpu.VMEM((chunk,), jnp.float32),
            pltpu.VMEM((4, chunk), jnp.float32),
            pltpu.VMEM((chunk,), jnp.int32),
            pltpu.VMEM((n_dma, 128), jnp.int32),
            pltpu.VMEM((chunk * 2, 128), jnp.float32),
            pltpu.VMEM((chunk, C), jnp.float32),
            pltpu.SemaphoreType.DMA,
        ],
        compiler_params=pltpu.CompilerParams(needs_layout_passes=False,
                                             use_tc_tiling_on_sc=False),
    )
    def run(knots_hbm, x_hbm, coef_hbm, out_hbm,
            knots_v, x_v, w_v, r_v, idx_v, rows_v, out_v, sem):
        cid = lax.axis_index("c")
        sid = lax.axis_index("s")
        wid = sid * NC + cid
        wbase = wid * rows_per_w
        lane = lax.iota(jnp.int32, _L)

        def chunk_body(c, carry):
            base = wbase + c * chunk
            pltpu.sync_copy(knots_hbm.at[pl.ds(base, chunk)], knots_v)
            pltpu.sync_copy(x_hbm.at[pl.ds(base, chunk)], x_v)

            def prep_body(g, carry2):
                lrow = g * _L + lane
                x = x_v[pl.ds(g * _L, _L)]
                cnt = jnp.zeros((_L,), jnp.int32)
                for i in range(n_knots):
                    t_i = plsc.load_gather(
                        knots_v, [lrow, jnp.full((_L,), i, jnp.int32)])
                    cnt = cnt + jnp.where(t_i <= x, 1, 0)
                k = jnp.clip(cnt - 1, 3, n_knots - 5)
                col = k - 2
                ts = [plsc.load_gather(knots_v, [lrow, col + m])
                      for m in range(6)]
                w = [[1.0 if cc == j else 0.0 for cc in range(4)]
                     for j in range(4)]
                for r in range(1, 4):
                    for j in range(3, r - 1, -1):
                        a = (x - ts[j - 1]) / (ts[j + 3 - r] - ts[j - 1])
                        om = 1.0 - a
                        w[j] = [_wsum(om, a, w[j - 1][cc], w[j][cc])
                                for cc in range(4)]
                for j in range(4):
                    w_v[j, pl.ds(g * _L, _L)] = w[3][j]
                g0 = (base + lrow) * n_coef + (k - 3)
                q = g0 >> 2
                q2 = jnp.minimum(q + 1, n_blk - 1)
                r_v[pl.ds(g * _L, _L)] = g0 & 3
                p = lrow * 2
                plsc.store_scatter(idx_v, [p >> 7, p & 127], q)
                p1 = p + 1
                plsc.store_scatter(idx_v, [p1 >> 7, p1 & 127], q2)
                return carry2

            lax.fori_loop(0, groups, prep_body, 0)

            cps = [pltpu.async_copy(coef_hbm.at[idx_v.at[i]],
                                    rows_v.at[pl.ds(i * 128, 128)], sem)
                   for i in range(n_dma)]
            for cp in cps:
                cp.wait()

            def sum_body(g, carry2):
                lrow = g * _L + lane
                rbase = lrow * 2
                ws = [w_v[j, pl.ds(g * _L, _L)] for j in range(4)]
                r = r_v[pl.ds(g * _L, _L)]
                ridx = []
                colb = []
                for j in range(4):
                    rj = r + j
                    ridx.append(rbase + (rj >> 2))
                    colb.append((rj & 3) * C)
                for ch in range(C):
                    chv = jnp.full((_L,), ch, jnp.int32)
                    acc = ws[0] * plsc.load_gather(
                        rows_v, [ridx[0], colb[0] + chv])
                    for j in range(1, 4):
                        acc = acc + ws[j] * plsc.load_gather(
                            rows_v, [ridx[j], colb[j] + chv])
                    plsc.store_scatter(out_v, [lrow, chv], acc)
                return carry2

            lax.fori_loop(0, groups, sum_body, 0)
            pltpu.sync_copy(out_v, out_hbm.at[pl.ds(base, chunk)])
            return carry

        lax.fori_loop(0, n_chunks, chunk_body, 0)

    return run


def kernel(coefficients, knots, inpce):
    B, n_coef, C = coefficients.shape
    n_knots = knots.shape[1]
    coef2 = coefficients.reshape(B * n_coef * C // 128, 128)
    x = inpce.reshape(B)
    run = _make_sc_eval(B, n_coef, C, n_knots)
    return run(knots, x, coef2)


# speedup vs baseline: 5.0639x; 5.0639x over previous
import functools

import jax
import jax.numpy as jnp
from jax import lax
from jax.experimental import pallas as pl
from jax.experimental.pallas import tpu as pltpu
from jax.experimental.pallas import tpu_sc as plsc

_L = 16


def _wsum(om, a, u, v):
    def term(s, w):
        if isinstance(w, float):
            if w == 0.0:
                return None
            if w == 1.0:
                return s
        return s * w
    t1, t2 = term(om, u), term(a, v)
    if t1 is None and t2 is None:
        return 0.0
    if t1 is None:
        return t2
    if t2 is None:
        return t1
    return t1 + t2


def _make_sc_eval(B, n_coef, C, n_knots):
    NC, NS = 2, 16
    NW = NC * NS
    assert B % NW == 0
    rows_per_w = B // NW
    chunk = 128
    assert rows_per_w % chunk == 0
    n_chunks = rows_per_w // chunk
    groups = chunk // _L
    kmax = n_knots - 5

    mesh = plsc.VectorSubcoreMesh(core_axis_name="c", subcore_axis_name="s",
                                  num_cores=NC, num_subcores=NS)

    @functools.partial(
        pl.kernel,
        out_type=jax.ShapeDtypeStruct((C, B), jnp.float32),
        mesh=mesh,
        scratch_types=[
            pltpu.VMEM((n_knots, chunk), jnp.float32),
            pltpu.VMEM((chunk,), jnp.float32),
            pltpu.VMEM((4, chunk), jnp.float32),
            pltpu.VMEM((chunk,), jnp.int32),
            pltpu.VMEM((n_coef, C, chunk), jnp.float32),
            pltpu.VMEM((C, chunk), jnp.float32),
            pltpu.SemaphoreType.DMA,
        ],
        compiler_params=pltpu.CompilerParams(needs_layout_passes=False,
                                             use_tc_tiling_on_sc=True),
    )
    def run(knots_hbm, x_hbm, coef_hbm, out_hbm,
            knots_v, x_v, w_v, km3_v, p_v, out_v, sem):
        cid = lax.axis_index("c")
        sid = lax.axis_index("s")
        wid = sid * NC + cid
        wbase = wid * rows_per_w
        lane = lax.iota(jnp.int32, _L)

        def chunk_body(c, carry):
            base = wbase + c * chunk
            cpc = pltpu.async_copy(
                coef_hbm.at[:, :, pl.ds(base, chunk)], p_v, sem)
            pltpu.sync_copy(knots_hbm.at[:, pl.ds(base, chunk)], knots_v)
            pltpu.sync_copy(x_hbm.at[pl.ds(base, chunk)], x_v)

            def prep_body(g, carry2):
                sl = pl.ds(g * _L, _L)
                x = x_v[sl]
                cnt = jnp.zeros((_L,), jnp.int32)
                for i in range(n_knots):
                    cnt = cnt + jnp.where(knots_v[i, sl] <= x, 1, 0)
                k = jnp.clip(cnt - 1, 3, kmax)
                col = g * _L + lane
                ts = [plsc.load_gather(knots_v, [k - 2 + m, col])
                      for m in range(6)]
                w = [[1.0 if cc == j else 0.0 for cc in range(4)]
                     for j in range(4)]
                for r in range(1, 4):
                    for j in range(3, r - 1, -1):
                        a = (x - ts[j - 1]) / (ts[j + 3 - r] - ts[j - 1])
                        om = 1.0 - a
                        w[j] = [_wsum(om, a, w[j - 1][cc], w[j][cc])
                                for cc in range(4)]
                for j in range(4):
                    w_v[j, sl] = w[3][j]
                km3_v[sl] = k - 3
                return carry2

            lax.fori_loop(0, groups, prep_body, 0)
            cpc.wait()

            def sum_body(g, carry2):
                sl = pl.ds(g * _L, _L)
                km3 = km3_v[sl]
                ws = [w_v[j, sl] for j in range(4)]
                acc = [None] * C
                for i in range(n_coef):
                    di = i - km3
                    wi = None
                    for j in range(max(0, i - (n_coef - 4)), min(3, i) + 1):
                        t = jnp.where(di == j, ws[j], 0.0)
                        wi = t if wi is None else wi + t
                    for ch in range(C):
                        v = wi * p_v[i, ch, sl]
                        acc[ch] = v if acc[ch] is None else acc[ch] + v
                for ch in range(C):
                    out_v[ch, sl] = acc[ch]
                return carry2

            lax.fori_loop(0, groups, sum_body, 0)
            pltpu.sync_copy(out_v, out_hbm.at[:, pl.ds(base, chunk)])
            return carry

        lax.fori_loop(0, n_chunks, chunk_body, 0)

    return run


def kernel(coefficients, knots, inpce):
    B, n_coef, C = coefficients.shape
    n_knots = knots.shape[1]
    coef_t = jnp.transpose(coefficients, (1, 2, 0))
    knots_t = knots.T
    x = inpce.reshape(B)
    run = _make_sc_eval(B, n_coef, C, n_knots)
    out_t = run(knots_t, x, coef_t)
    return out_t.T
